# baseline (speedup 1.0000x reference)
import functools

import jax
import jax.numpy as jnp
from jax import lax
from jax.experimental import pallas as pl
from jax.experimental.pallas import tpu as pltpu
from jax.experimental.pallas import tpu_sc as plsc

NC, NS, L = 2, 16, 16
NW = NC * NS
B = 16384
D = 32
BPW = B // NW
CH = 128
NCH = BPW // CH

_mesh = plsc.VectorSubcoreMesh(core_axis_name="c", subcore_axis_name="s")


@functools.partial(
    pl.kernel,
    out_type=jax.ShapeDtypeStruct((B,), jnp.float32),
    mesh=_mesh,
    compiler_params=pltpu.CompilerParams(
        needs_layout_passes=False, use_tc_tiling_on_sc=False),
    scratch_types=[
        pltpu.VMEM((NCH, CH, 2), jnp.int32),
        pltpu.VMEM((NCH, CH), jnp.int32),
        pltpu.VMEM((NCH, CH), jnp.int32),
        pltpu.VMEM((NCH, CH, D), jnp.float32),
        pltpu.VMEM((NCH, CH, D), jnp.float32),
        pltpu.VMEM((BPW,), jnp.float32),
        pltpu.SemaphoreType.DMA,
    ],
)
def _mf_sc(x_hbm, w_hbm, h_hbm, out_hbm,
           xr_v, ui_v, vi_v, ur_v, vr_v, o_v, sem):
    wid = lax.axis_index("s") * NC + lax.axis_index("c")
    base = wid * BPW

    for k in range(NCH):
        pltpu.sync_copy(x_hbm.at[pl.ds(base + k * CH, CH)], xr_v.at[k])

    lane = lax.iota(jnp.int32, L)
    zero = jnp.zeros((L,), jnp.int32)
    for k in range(NCH):
        for t in range(CH // L):
            rows = lane + t * L
            ui_v[k, pl.ds(t * L, L)] = plsc.load_gather(
                xr_v.at[k], [rows, zero])
            vi_v[k, pl.ds(t * L, L)] = plsc.load_gather(
                xr_v.at[k], [rows, zero + 1])

    copies = []
    for k in range(NCH):
        copies.append(pltpu.async_copy(w_hbm.at[ui_v.at[k]], ur_v.at[k], sem))
        copies.append(pltpu.async_copy(h_hbm.at[vi_v.at[k]], vr_v.at[k], sem))
    for c in copies:
        c.wait()

    def chunk_body(k, _):
        def group_body(g, _):
            acc = jnp.zeros((L,), jnp.float32)
            for j in range(L):
                r = g * L + j
                u0 = ur_v[k, r, pl.ds(0, L)]
                u1 = ur_v[k, r, pl.ds(L, L)]
                v0 = vr_v[k, r, pl.ds(0, L)]
                v1 = vr_v[k, r, pl.ds(L, L)]
                s = jnp.sum(u0 * v0 + u1 * v1)
                acc = jnp.where(lane == j, s, acc)
            o_v[pl.ds(k * CH + g * L, L)] = 1.0 / (1.0 + jnp.exp(-acc))
            return 0
        return lax.fori_loop(0, CH // L, group_body, 0)

    lax.fori_loop(0, NCH, chunk_body, 0)
    pltpu.sync_copy(o_v, out_hbm.at[pl.ds(base, BPW)])


def kernel(x, W, H):
    return _mf_sc(x.astype(jnp.int32), W, H)


# speedup vs baseline: 1.3823x; 1.3823x over previous
import functools

import jax
import jax.numpy as jnp
from jax import lax
from jax.experimental import pallas as pl
from jax.experimental.pallas import tpu as pltpu
from jax.experimental.pallas import tpu_sc as plsc

NC, NS, L = 2, 16, 16
NW = NC * NS
B = 16384
D = 32
BPW = B // NW
CH = 128
NCH = BPW // CH
NBUF = 2

_mesh = plsc.VectorSubcoreMesh(core_axis_name="c", subcore_axis_name="s")


@functools.partial(
    pl.kernel,
    out_type=jax.ShapeDtypeStruct((B,), jnp.float32),
    mesh=_mesh,
    compiler_params=pltpu.CompilerParams(
        needs_layout_passes=False, use_tc_tiling_on_sc=True),
    scratch_types=[
        pltpu.VMEM((NCH, CH), jnp.int32),
        pltpu.VMEM((NCH, CH), jnp.int32),
        pltpu.VMEM((NBUF, CH, D), jnp.float32),
        pltpu.VMEM((NBUF, CH, D), jnp.float32),
        pltpu.VMEM((BPW,), jnp.float32),
        pltpu.SemaphoreType.DMA,
        pltpu.SemaphoreType.DMA,
    ],
)
def _mf_sc(u_hbm, v_hbm, w_hbm, h_hbm, out_hbm,
           u_s, v_s, wbuf, hbuf, o_v, sem0, sem1):
    wid = lax.axis_index("s") * NC + lax.axis_index("c")
    base = wid * BPW

    for k in range(NCH):
        pltpu.sync_copy(u_hbm.at[pl.ds(base + k * CH, CH)], u_s.at[k])
        pltpu.sync_copy(v_hbm.at[pl.ds(base + k * CH, CH)], v_s.at[k])

    sems = (sem0, sem1)

    def issue(k):
        b = k % NBUF
        sem = sems[b]

        def issue_body(g, _, k=k, b=b, sem=sem):
            u16 = u_s[k, pl.ds(g * L, L)]
            v16 = v_s[k, pl.ds(g * L, L)]
            for j in range(L):
                us = u16[j]
                vs = v16[j]
                pltpu.async_copy(w_hbm.at[pl.ds(us, 1)],
                                 wbuf.at[b, pl.ds(g * L + j, 1)], sem)
                pltpu.async_copy(h_hbm.at[pl.ds(vs, 1)],
                                 hbuf.at[b, pl.ds(g * L + j, 1)], sem)
            return 0

        lax.fori_loop(0, CH // L, issue_body, 0)

    def drain(k):
        b = k % NBUF
        sem = sems[b]
        pltpu.make_async_copy(w_hbm.at[pl.ds(0, CH)], wbuf.at[b], sem).wait()
        pltpu.make_async_copy(h_hbm.at[pl.ds(0, CH)], hbuf.at[b], sem).wait()

    lane = lax.iota(jnp.int32, L)
    issue(0)

    for k in range(NCH):
        if k + 1 < NCH:
            issue(k + 1)
        drain(k)
        b = k % NBUF

        def group_body(g, _, k=k, b=b):
            acc = jnp.zeros((L,), jnp.float32)
            for j in range(L):
                r = g * L + j
                u0 = wbuf[b, r, pl.ds(0, L)]
                u1 = wbuf[b, r, pl.ds(L, L)]
                v0 = hbuf[b, r, pl.ds(0, L)]
                v1 = hbuf[b, r, pl.ds(L, L)]
                s = jnp.sum(u0 * v0 + u1 * v1)
                acc = jnp.where(lane == j, s, acc)
            o_v[pl.ds(k * CH + g * L, L)] = 1.0 / (1.0 + jnp.exp(-acc))
            return 0

        lax.fori_loop(0, CH // L, group_body, 0)

    pltpu.sync_copy(o_v, out_hbm.at[pl.ds(base, BPW)])


def kernel(x, W, H):
    xi = x.astype(jnp.int32)
    return _mf_sc(xi[:, 0], xi[:, 1], W, H)


# speedup vs baseline: 1.8957x; 1.3714x over previous
import functools

import jax
import jax.numpy as jnp
from jax import lax
from jax.experimental import pallas as pl
from jax.experimental.pallas import tpu as pltpu
from jax.experimental.pallas import tpu_sc as plsc

NC, NS, L = 2, 16, 16
NW = NC * NS
B = 16384
D = 32
NR = 100000
SR = NR // 8
BPW = B // NW
CH = 128
CH8 = CH // 8
NCH = BPW // CH
NBUF = 2

_mesh = plsc.VectorSubcoreMesh(core_axis_name="c", subcore_axis_name="s")


@functools.partial(
    pl.kernel,
    out_type=jax.ShapeDtypeStruct((B,), jnp.float32),
    mesh=_mesh,
    compiler_params=pltpu.CompilerParams(
        needs_layout_passes=False, use_tc_tiling_on_sc=True),
    scratch_types=[
        pltpu.VMEM((NCH, CH), jnp.int32),
        pltpu.VMEM((NCH, CH), jnp.int32),
        pltpu.VMEM((NBUF, CH8, 8, D), jnp.float32),
        pltpu.VMEM((NBUF, CH8, 8, D), jnp.float32),
        pltpu.VMEM((BPW,), jnp.float32),
        pltpu.SemaphoreType.DMA,
        pltpu.SemaphoreType.DMA,
    ],
)
def _mf_sc(u_hbm, v_hbm, w_hbm, h_hbm, out_hbm,
           u_s, v_s, wbuf, hbuf, o_v, sem0, sem1):
    wid = lax.axis_index("s") * NC + lax.axis_index("c")
    base = wid * BPW

    for k in range(NCH):
        pltpu.sync_copy(u_hbm.at[pl.ds(base + k * CH, CH)], u_s.at[k])
        pltpu.sync_copy(v_hbm.at[pl.ds(base + k * CH, CH)], v_s.at[k])

    sems = (sem0, sem1)

    def issue(k):
        b = k % NBUF
        sem = sems[b]

        def issue_body(g, _, k=k, b=b, sem=sem):
            u16 = u_s[k, pl.ds(g * L, L)]
            v16 = v_s[k, pl.ds(g * L, L)]
            for j in range(L):
                us = u16[j]
                vs = v16[j]
                q = g * 2 + j // 8
                pltpu.async_copy(w_hbm.at[us >> 3, pl.ds(us & 7, 1)],
                                 wbuf.at[b, q, pl.ds(j & 7, 1)], sem)
                pltpu.async_copy(h_hbm.at[vs >> 3, pl.ds(vs & 7, 1)],
                                 hbuf.at[b, q, pl.ds(j & 7, 1)], sem)
            return 0

        lax.fori_loop(0, CH // L, issue_body, 0)

    def drain(k):
        b = k % NBUF
        sem = sems[b]
        pltpu.make_async_copy(w_hbm.at[pl.ds(0, CH8)], wbuf.at[b], sem).wait()
        pltpu.make_async_copy(h_hbm.at[pl.ds(0, CH8)], hbuf.at[b], sem).wait()

    lane = lax.iota(jnp.int32, L)
    issue(0)

    for k in range(NCH):
        if k + 1 < NCH:
            issue(k + 1)
        drain(k)
        b = k % NBUF

        def group_body(g, _, k=k, b=b):
            acc = jnp.zeros((L,), jnp.float32)
            for j in range(L):
                q = g * 2 + j // 8
                s = j & 7
                u0 = wbuf[b, q, s, pl.ds(0, L)]
                u1 = wbuf[b, q, s, pl.ds(L, L)]
                v0 = hbuf[b, q, s, pl.ds(0, L)]
                v1 = hbuf[b, q, s, pl.ds(L, L)]
                t = jnp.sum(u0 * v0 + u1 * v1)
                acc = jnp.where(lane == j, t, acc)
            o_v[pl.ds(k * CH + g * L, L)] = 1.0 / (1.0 + jnp.exp(-acc))
            return 0

        lax.fori_loop(0, CH // L, group_body, 0)

    pltpu.sync_copy(o_v, out_hbm.at[pl.ds(base, BPW)])


def kernel(x, W, H):
    xi = x.astype(jnp.int32)
    Wc = W.reshape(SR, 8, D)
    Hc = H.reshape(SR, 8, D)
    return _mf_sc(xi[:, 0], xi[:, 1], Wc, Hc)


# speedup vs baseline: 2.0080x; 1.0592x over previous
import functools

import jax
import jax.numpy as jnp
from jax import lax
from jax.experimental import pallas as pl
from jax.experimental.pallas import tpu as pltpu
from jax.experimental.pallas import tpu_sc as plsc

NC, NS, L = 2, 16, 16
NW = NC * NS
B = 16384
D = 32
NR = 100000
SR = NR // 8
BPW = B // NW
CH = 128
CH8 = CH // 8
NCH = BPW // CH
NBUF = 2

_mesh = plsc.VectorSubcoreMesh(core_axis_name="c", subcore_axis_name="s")


@functools.partial(
    pl.kernel,
    out_type=jax.ShapeDtypeStruct((B,), jnp.float32),
    mesh=_mesh,
    compiler_params=pltpu.CompilerParams(
        needs_layout_passes=False, use_tc_tiling_on_sc=True),
    scratch_types=[
        pltpu.VMEM((BPW,), jnp.int32),
        pltpu.VMEM((BPW,), jnp.int32),
        pltpu.VMEM((NBUF, CH8, 8, D), jnp.float32),
        pltpu.VMEM((NBUF, CH8, 8, D), jnp.float32),
        pltpu.VMEM((BPW,), jnp.float32),
        pltpu.SemaphoreType.DMA,
        pltpu.SemaphoreType.DMA,
    ],
)
def _mf_sc(u_hbm, v_hbm, w_hbm, h_hbm, out_hbm,
           u_s, v_s, wbuf, hbuf, o_v, sem0, sem1):
    wid = lax.axis_index("s") * NC + lax.axis_index("c")
    base = wid * BPW

    cu = pltpu.async_copy(u_hbm.at[pl.ds(base, BPW)], u_s, sem0)
    cv = pltpu.async_copy(v_hbm.at[pl.ds(base, BPW)], v_s, sem1)
    cu.wait()
    cv.wait()

    sems = (sem0, sem1)

    def issue(k):
        b = k % NBUF
        sem = sems[b]

        def issue_body(g, _, k=k, b=b, sem=sem):
            u16 = u_s[pl.ds(k * CH + g * L, L)]
            v16 = v_s[pl.ds(k * CH + g * L, L)]
            for j in range(L):
                us = u16[j]
                vs = v16[j]
                q = g * 2 + j // 8
                pltpu.async_copy(w_hbm.at[us >> 3, pl.ds(us & 7, 1)],
                                 wbuf.at[b, q, pl.ds(j & 7, 1)], sem)
                pltpu.async_copy(h_hbm.at[vs >> 3, pl.ds(vs & 7, 1)],
                                 hbuf.at[b, q, pl.ds(j & 7, 1)], sem)
            return 0

        lax.fori_loop(0, CH // L, issue_body, 0)

    def drain(k):
        b = k % NBUF
        sem = sems[b]
        pltpu.make_async_copy(w_hbm.at[pl.ds(0, CH8)], wbuf.at[b], sem).wait()
        pltpu.make_async_copy(h_hbm.at[pl.ds(0, CH8)], hbuf.at[b], sem).wait()

    lane = lax.iota(jnp.int32, L)
    issue(0)

    for k in range(NCH):
        if k + 1 < NCH:
            issue(k + 1)
        drain(k)
        b = k % NBUF

        def group_body(g, _, k=k, b=b):
            acc = jnp.zeros((L,), jnp.float32)
            for j in range(L):
                q = g * 2 + j // 8
                s = j & 7
                u0 = wbuf[b, q, s, pl.ds(0, L)]
                u1 = wbuf[b, q, s, pl.ds(L, L)]
                v0 = hbuf[b, q, s, pl.ds(0, L)]
                v1 = hbuf[b, q, s, pl.ds(L, L)]
                t = jnp.sum(u0 * v0 + u1 * v1)
                acc = jnp.where(lane == j, t, acc)
            o_v[pl.ds(k * CH + g * L, L)] = 1.0 / (1.0 + jnp.exp(-acc))
            return 0

        lax.fori_loop(0, CH // L, group_body, 0)

    pltpu.sync_copy(o_v, out_hbm.at[pl.ds(base, BPW)])


def kernel(x, W, H):
    xi = x.astype(jnp.int32)
    Wc = W.reshape(SR, 8, D)
    Hc = H.reshape(SR, 8, D)
    return _mf_sc(xi[:, 0], xi[:, 1], Wc, Hc)
